# Initial kernel scaffold; baseline (speedup 1.0000x reference)
import jax
import jax.numpy as jnp
from jax.experimental import pallas as pl

T, B, C, H, FFN, VQD, K = 2048, 1, 1024, 16, 4096, 256, 128
D = C // H
COMMITMENT = 1.0


def _qkv_kernel(x_ref, w_ref, b_ref, out_ref):
    out_ref[...] = x_ref[...] @ w_ref[...] + b_ref[...]


def _attn_kernel(q_ref, k_ref, v_ref, o_ref):
    q = q_ref[...]
    k = k_ref[...]
    s = jax.lax.dot_general(q, k, (((1,), (1,)), ((), ())))
    p = jax.nn.softmax(s, axis=-1)
    o_ref[...] = p @ v_ref[...]


def _ln(y, g, b):
    m = jnp.mean(y, axis=-1, keepdims=True)
    v = jnp.mean((y - m) ** 2, axis=-1, keepdims=True)
    return (y - m) * jax.lax.rsqrt(v + 1e-5) * g + b


def _vq_kernel(o_ref, x_ref, wo_ref, bo_ref, g1_ref, b1_ref, wtovq_ref,
               cb_ref, wtoemb_ref, gv_ref, bv_ref, m_ref,
               x2_ref, loss_ref):
    x1 = _ln(x_ref[...] + o_ref[...] @ wo_ref[...] + bo_ref[...],
             g1_ref[...], b1_ref[...])
    flat = x1 @ wtovq_ref[...]
    cb = cb_ref[...]
    d2 = (-2.0) * jax.lax.dot_general(flat, cb, (((1,), (1,)), ((), ()))) \
        + jnp.sum(cb * cb, axis=1)[None, :]
    mins = jnp.min(d2, axis=1, keepdims=True)
    iota = jax.lax.broadcasted_iota(jnp.int32, d2.shape, 1)
    idx = jnp.min(jnp.where(d2 == mins, iota, K), axis=1, keepdims=True)
    oh = (iota == idx).astype(jnp.float32)
    quant = oh @ cb
    m = m_ref[...]
    diff = quant - flat
    per_tok = jnp.sum(diff * diff, axis=1, keepdims=True) * (1.0 / VQD)
    num = jnp.sum(per_tok * m)
    den = jnp.maximum(jnp.sum(m), 1.0)
    loss_ref[0, 0] = COMMITMENT * num / den
    table = cb @ wtoemb_ref[...]
    eca = (oh @ table) * m
    x2 = _ln(x1 + eca, gv_ref[...], bv_ref[...])
    x2_ref[...] = jnp.where(m > 0.0, x2, x1)


def _ffn_kernel(x_ref, w1_ref, b1_ref, w2_ref, b2_ref, g_ref, b_ref, out_ref):
    xb = x_ref[...]
    h = jax.nn.relu(xb @ w1_ref[...] + b1_ref[...])
    y = xb + h @ w2_ref[...] + b2_ref[...]
    out_ref[...] = _ln(y, g_ref[...], b_ref[...])


def kernel(x, quantization_mask, Wq, bq, Wk, bk, Wv, bv, Wo, bo, ln1_g, ln1_b,
           Wtovq, codebook, Wtoemb, lnvq_g, lnvq_b, W1, b1, W2, b2, ln2_g, ln2_b):
    x2d = x.reshape(T, C)
    scale = D ** -0.5
    wqkv = jnp.concatenate([Wq * scale, Wk, Wv], axis=1)
    bqkv = jnp.concatenate([bq * scale, bk, bv]).reshape(1, 3 * C)
    qkv = pl.pallas_call(
        _qkv_kernel,
        out_shape=jax.ShapeDtypeStruct((T, 3 * C), jnp.float32),
    )(x2d, wqkv, bqkv)

    attn_o = pl.pallas_call(
        _attn_kernel,
        grid=(H,),
        in_specs=[
            pl.BlockSpec((T, D), lambda h: (0, h)),
            pl.BlockSpec((T, D), lambda h: (0, H + h)),
            pl.BlockSpec((T, D), lambda h: (0, 2 * H + h)),
        ],
        out_specs=pl.BlockSpec((T, D), lambda h: (0, h)),
        out_shape=jax.ShapeDtypeStruct((T, C), jnp.float32),
    )(qkv, qkv, qkv)

    mask_col = quantization_mask.reshape(T, 1).astype(jnp.float32)
    x2, loss = pl.pallas_call(
        _vq_kernel,
        out_shape=(
            jax.ShapeDtypeStruct((T, C), jnp.float32),
            jax.ShapeDtypeStruct((1, 1), jnp.float32),
        ),
    )(attn_o, x2d, Wo, bo.reshape(1, C), ln1_g.reshape(1, C),
      ln1_b.reshape(1, C), Wtovq, codebook, Wtoemb,
      lnvq_g.reshape(1, C), lnvq_b.reshape(1, C), mask_col)

    RT = 512
    x3 = pl.pallas_call(
        _ffn_kernel,
        grid=(T // RT,),
        in_specs=[
            pl.BlockSpec((RT, C), lambda i: (i, 0)),
            pl.BlockSpec((C, FFN), lambda i: (0, 0)),
            pl.BlockSpec((1, FFN), lambda i: (0, 0)),
            pl.BlockSpec((FFN, C), lambda i: (0, 0)),
            pl.BlockSpec((1, C), lambda i: (0, 0)),
            pl.BlockSpec((1, C), lambda i: (0, 0)),
            pl.BlockSpec((1, C), lambda i: (0, 0)),
        ],
        out_specs=pl.BlockSpec((RT, C), lambda i: (i, 0)),
        out_shape=jax.ShapeDtypeStruct((T, C), jnp.float32),
    )(x2, W1, b1.reshape(1, FFN), W2, b2.reshape(1, C),
      ln2_g.reshape(1, C), ln2_b.reshape(1, C))

    return x3.reshape(T, B, C), loss[0, 0]


# speedup vs baseline: 1.8023x; 1.8023x over previous
import jax
import jax.numpy as jnp
from jax.experimental import pallas as pl

T, B, C, H, FFN, VQD, K = 2048, 1, 1024, 16, 4096, 256, 128
D = C // H
COMMITMENT = 1.0


def _qkv_kernel(x_ref, w_ref, b_ref, out_ref):
    out_ref[...] = x_ref[...] @ w_ref[...] + b_ref[...]


def _attn_kernel(q_ref, k_ref, v_ref, o_ref):
    for i in range(2):
        sl = slice(i * D, (i + 1) * D)
        q = q_ref[:, sl]
        k = k_ref[:, sl]
        s = jax.lax.dot_general(q, k, (((1,), (1,)), ((), ())))
        p = jax.nn.softmax(s, axis=-1)
        o_ref[:, sl] = p @ v_ref[:, sl]


def _ln(y, g, b):
    m = jnp.mean(y, axis=-1, keepdims=True)
    v = jnp.mean((y - m) ** 2, axis=-1, keepdims=True)
    return (y - m) * jax.lax.rsqrt(v + 1e-5) * g + b


def _vq_kernel(o_ref, x_ref, wo_ref, bo_ref, g1_ref, b1_ref, wtovq_ref,
               cb_ref, wtoemb_ref, gv_ref, bv_ref, m_ref,
               x2_ref, loss_ref):
    x1 = _ln(x_ref[...] + o_ref[...] @ wo_ref[...] + bo_ref[...],
             g1_ref[...], b1_ref[...])
    flat = x1 @ wtovq_ref[...]
    cb = cb_ref[...]
    d2 = (-2.0) * jax.lax.dot_general(flat, cb, (((1,), (1,)), ((), ()))) \
        + jnp.sum(cb * cb, axis=1)[None, :]
    mins = jnp.min(d2, axis=1, keepdims=True)
    iota = jax.lax.broadcasted_iota(jnp.int32, d2.shape, 1)
    idx = jnp.min(jnp.where(d2 == mins, iota, K), axis=1, keepdims=True)
    oh = (iota == idx).astype(jnp.float32)
    quant = oh @ cb
    m = m_ref[...]
    diff = quant - flat
    per_tok = jnp.sum(diff * diff, axis=1, keepdims=True) * (1.0 / VQD)
    num = jnp.sum(per_tok * m)
    den = jnp.maximum(jnp.sum(m), 1.0)
    loss_ref[...] = jnp.reshape(COMMITMENT * num / den, (1, 1))
    table = cb @ wtoemb_ref[...]
    eca = (oh @ table) * m
    x2 = _ln(x1 + eca, gv_ref[...], bv_ref[...])
    x2_ref[...] = jnp.where(m > 0.0, x2, x1)


def _ffn_kernel(x_ref, w1_ref, b1_ref, w2_ref, b2_ref, g_ref, b_ref, out_ref):
    xb = x_ref[...]
    h = jax.nn.relu(xb @ w1_ref[...] + b1_ref[...])
    y = xb + h @ w2_ref[...] + b2_ref[...]
    out_ref[...] = _ln(y, g_ref[...], b_ref[...])


def kernel(x, quantization_mask, Wq, bq, Wk, bk, Wv, bv, Wo, bo, ln1_g, ln1_b,
           Wtovq, codebook, Wtoemb, lnvq_g, lnvq_b, W1, b1, W2, b2, ln2_g, ln2_b):
    x2d = x.reshape(T, C)
    scale = D ** -0.5
    wqkv = jnp.concatenate([Wq * scale, Wk, Wv], axis=1)
    bqkv = jnp.concatenate([bq * scale, bk, bv]).reshape(1, 3 * C)
    qkv = pl.pallas_call(
        _qkv_kernel,
        out_shape=jax.ShapeDtypeStruct((T, 3 * C), jnp.float32),
    )(x2d, wqkv, bqkv)

    attn_o = pl.pallas_call(
        _attn_kernel,
        grid=(H // 2,),
        in_specs=[
            pl.BlockSpec((T, 2 * D), lambda h: (0, h)),
            pl.BlockSpec((T, 2 * D), lambda h: (0, H // 2 + h)),
            pl.BlockSpec((T, 2 * D), lambda h: (0, H + h)),
        ],
        out_specs=pl.BlockSpec((T, 2 * D), lambda h: (0, h)),
        out_shape=jax.ShapeDtypeStruct((T, C), jnp.float32),
    )(qkv, qkv, qkv)

    mask_col = quantization_mask.reshape(T, 1).astype(jnp.float32)
    x2, loss = pl.pallas_call(
        _vq_kernel,
        out_shape=(
            jax.ShapeDtypeStruct((T, C), jnp.float32),
            jax.ShapeDtypeStruct((1, 1), jnp.float32),
        ),
    )(attn_o, x2d, Wo, bo.reshape(1, C), ln1_g.reshape(1, C),
      ln1_b.reshape(1, C), Wtovq, codebook, Wtoemb,
      lnvq_g.reshape(1, C), lnvq_b.reshape(1, C), mask_col)

    RT = 512
    x3 = pl.pallas_call(
        _ffn_kernel,
        grid=(T // RT,),
        in_specs=[
            pl.BlockSpec((RT, C), lambda i: (i, 0)),
            pl.BlockSpec((C, FFN), lambda i: (0, 0)),
            pl.BlockSpec((1, FFN), lambda i: (0, 0)),
            pl.BlockSpec((FFN, C), lambda i: (0, 0)),
            pl.BlockSpec((1, C), lambda i: (0, 0)),
            pl.BlockSpec((1, C), lambda i: (0, 0)),
            pl.BlockSpec((1, C), lambda i: (0, 0)),
        ],
        out_specs=pl.BlockSpec((RT, C), lambda i: (i, 0)),
        out_shape=jax.ShapeDtypeStruct((T, C), jnp.float32),
    )(x2, W1, b1.reshape(1, FFN), W2, b2.reshape(1, C),
      ln2_g.reshape(1, C), ln2_b.reshape(1, C))

    return x3.reshape(T, B, C), loss[0, 0]
